# baseline (speedup 1.0000x reference)
import jax
import jax.numpy as jnp
from jax import lax
from jax.experimental import pallas as pl
from jax.experimental.pallas import tpu as pltpu

_EPS = 1e-8
_G = 8


def _dtw_body(x_ref, y_ref, out_ref, st_ref, cs_ref, skb_ref, sm_ref, dsem):
    G = x_ref.shape[1]
    N = x_ref.shape[2]
    M = y_ref.shape[2]
    K = N + M - 1
    K2 = skb_ref.shape[0]
    LW = skb_ref.shape[2]
    CH = N // LW
    INF = jnp.float32(jnp.inf)

    x = x_ref[0]
    y = y_ref[0]
    xn = x / jnp.maximum(jnp.sqrt(jnp.sum(x * x, axis=2, keepdims=True)), _EPS)
    yn = y / jnp.maximum(jnp.sqrt(jnp.sum(y * y, axis=2, keepdims=True)), _EPS)
    costT = 1.0 - lax.dot_general(
        yn, xn, (((2,), (2,)), ((0,), (0,))),
        preferred_element_type=jnp.float32,
    )

    s_all = jnp.sum(costT, axis=2)
    mneg = jnp.max(s_all, axis=1, keepdims=True)
    neg_v = mneg + jnp.log(
        jnp.sum(jnp.exp(s_all - mneg), axis=1, keepdims=True)
    )

    lane2 = lax.broadcasted_iota(jnp.int32, (K2, N), 1)
    for g in range(G):
        buf = jnp.concatenate(
            [costT[g], jnp.full((K2 - M, N), INF, jnp.float32)], axis=0
        )
        shift = 1
        while shift < N:
            rolled = pltpu.roll(buf, shift, 0)
            buf = jnp.where((lane2 & shift) != 0, rolled, buf)
            shift *= 2
        skb_ref[:, CH * g : CH * (g + 1), :] = buf.reshape(K2, CH, LW)

    lane1 = lax.broadcasted_iota(jnp.int32, (G, N), 1)

    def shift1(p):
        r = pltpu.roll(p, 1, 1)
        return jnp.where(lane1 == 0, INF, r)

    def page(k_dyn):
        return skb_ref[pl.ds(k_dyn, 1), :, :].reshape(G, N)

    cd0 = page(0)
    p1 = jnp.where(lane1 == 0, cd0, INF)
    st_ref[0] = p1
    st_ref[1] = shift1(p1)
    st_ref[2] = jnp.full((G, N), INF, jnp.float32)

    def dp_body(k, carry):
        cd = page(k)
        c = st_ref[0]
        b = st_ref[pl.ds(1 + lax.rem(k + 1, 2), 1)].reshape(G, N)
        a_pg = 1 + lax.rem(k, 2)
        a = st_ref[pl.ds(a_pg, 1)].reshape(G, N)
        mbc = jnp.minimum(b, c)
        v = cd + jnp.minimum(a, mbc)
        diag = a <= mbc
        up = jnp.logical_not(diag) & (b <= c)
        code = jnp.where(diag, 0.0, jnp.where(up, 4.0, 8.0))
        skb_ref[pl.ds(k, 1), :, :] = (cd + code).reshape(1, CH * G, LW)
        st_ref[0] = v
        st_ref[pl.ds(a_pg, 1)] = shift1(v).reshape(1, G, N)
        return carry

    lax.fori_loop(1, K, dp_body, 0)

    laneM = lax.broadcasted_iota(jnp.int32, (1, M), 1)
    skew00 = [skb_ref[0, CH * g, 0] for g in range(G)]

    def _dec(e):
        return (
            e
            - 4.0 * (e >= 3.0).astype(jnp.float32)
            - 4.0 * (e >= 7.0).astype(jnp.float32)
        )

    c_last = [
        _dec(skb_ref[K - 1, CH * g + CH - 1, LW - 1]) for g in range(G)
    ]

    cs_ref[...] = jnp.zeros((G, M), jnp.float32)
    zcopy = pltpu.make_async_copy(cs_ref, sm_ref, dsem)
    zcopy.start()
    zcopy.wait()
    for g in range(G):
        sm_ref[g, M - 1] = c_last[g]

    def read_enc(g, k, i):
        sub = CH * g + lax.div(i, LW)
        off = lax.rem(i, LW)
        row = skb_ref[pl.ds(k, 1), pl.ds(sub, 1), :]
        amt = lax.rem(jnp.int32(LW) - off, jnp.int32(LW))
        return pltpu.roll(row, amt, 2)[0, 0, 0]

    iN = jnp.int32(N - 1)
    jM = jnp.int32(M - 1)
    enc0 = [read_enc(g, K - 1, iN) for g in range(G)]
    ij0 = iN * jnp.int32(1024) + jM
    init = tuple([ij0] * G + enc0 + c_last)

    def bt_body(t, carry):
        ijs = list(carry[:G])
        encs = list(carry[G : 2 * G])
        accs = list(carry[2 * G :])
        for g in range(G):
            ij, enc, acc = ijs[g], encs[g], accs[g]
            i = lax.shift_right_logical(ij, 10)
            j = lax.bitwise_and(ij, jnp.int32(1023))
            pred = (i > 0) & (j > 0)
            ge4 = enc >= 3.0
            ge8 = enc >= 7.0
            go_i = pred & jnp.logical_not(ge8)
            go_j = pred & (jnp.logical_not(ge4) | ge8)
            nij = (
                ij
                - jnp.where(go_i, jnp.int32(1024), jnp.int32(0))
                - jnp.where(go_j, jnp.int32(1), jnp.int32(0))
            )
            ni = lax.shift_right_logical(nij, 10)
            nj = lax.bitwise_and(nij, jnp.int32(1023))
            kn = ni + nj
            enc_n = jnp.where(kn == 0, skew00[g], read_enc(g, kn, ni))
            cost_n = _dec(enc_n)
            acc = jnp.where(
                pred, jnp.where(go_j, cost_n, acc + cost_n), acc
            )
            sm_ref[g, nj] = acc
            ijs[g], encs[g], accs[g] = nij, enc_n, acc
        return tuple(ijs + encs + accs)

    fin = lax.fori_loop(0, K - 1, bt_body, init)

    for g in range(G):
        both0 = fin[g] == 0
        sm_ref[g, 0] = jnp.where(
            both0, sm_ref[g, 0], sm_ref[g, 0] + skew00[g]
        )

    bcopy = pltpu.make_async_copy(sm_ref, cs_ref, dsem)
    bcopy.start()
    bcopy.wait()
    cs = cs_ref[...]
    mpos = jnp.max(cs, axis=1, keepdims=True)
    pos_v = mpos + jnp.log(
        jnp.sum(jnp.exp(cs - mpos), axis=1, keepdims=True)
    )
    out_ref[0] = jnp.broadcast_to(pos_v - neg_v, (G, 128))


def _dtw_pallas(x, y, interpret=False):
    B, N, D = x.shape
    M = y.shape[1]
    G = _G
    assert B % G == 0 and N < 1024 and M < 1024
    K2 = ((N + M - 1) + 7) // 8 * 8
    LW = 128 if N % 128 == 0 else N
    CH = N // LW
    xg = x.reshape(B // G, G, N, D)
    yg = y.reshape(B // G, G, M, D)
    out = pl.pallas_call(
        _dtw_body,
        grid=(B // G,),
        in_specs=[
            pl.BlockSpec((1, G, N, D), lambda b: (b, 0, 0, 0)),
            pl.BlockSpec((1, G, M, D), lambda b: (b, 0, 0, 0)),
        ],
        out_specs=pl.BlockSpec((1, G, 128), lambda b: (b, 0, 0)),
        out_shape=jax.ShapeDtypeStruct((B // G, G, 128), jnp.float32),
        scratch_shapes=[
            pltpu.VMEM((3, G, N), jnp.float32),
            pltpu.VMEM((G, M), jnp.float32),
            pltpu.VMEM((K2, CH * G, LW), jnp.float32),
            pltpu.SMEM((G, M), jnp.float32),
            pltpu.SemaphoreType.DMA,
        ],
        compiler_params=pltpu.CompilerParams(
            dimension_semantics=("parallel",),
            vmem_limit_bytes=48 * 1024 * 1024,
        ),
        interpret=interpret,
    )(xg, yg)
    return out[:, :, 0].reshape(B)


def kernel(x, y):
    return _dtw_pallas(x, y)


# speedup vs baseline: 1.1262x; 1.1262x over previous
import jax
import jax.numpy as jnp
from jax import lax
from jax.experimental import pallas as pl
from jax.experimental.pallas import tpu as pltpu

_EPS = 1e-8
_G = 8


def _dtw_body(x_ref, y_ref, out_ref, st_ref, cs_ref, skb_ref):
    G = x_ref.shape[1]
    N = x_ref.shape[2]
    M = y_ref.shape[2]
    K = N + M - 1
    K2 = skb_ref.shape[0]
    LW = skb_ref.shape[2]
    CH = N // LW
    INF = jnp.float32(jnp.inf)

    x = x_ref[0]
    y = y_ref[0]
    xn = x / jnp.maximum(jnp.sqrt(jnp.sum(x * x, axis=2, keepdims=True)), _EPS)
    yn = y / jnp.maximum(jnp.sqrt(jnp.sum(y * y, axis=2, keepdims=True)), _EPS)
    costT = 1.0 - lax.dot_general(
        yn, xn, (((2,), (2,)), ((0,), (0,))),
        preferred_element_type=jnp.float32,
    )

    s_all = jnp.sum(costT, axis=2)
    mneg = jnp.max(s_all, axis=1, keepdims=True)
    neg_v = mneg + jnp.log(
        jnp.sum(jnp.exp(s_all - mneg), axis=1, keepdims=True)
    )

    lane2 = lax.broadcasted_iota(jnp.int32, (K2, N), 1)
    for g in range(G):
        buf = jnp.concatenate(
            [costT[g], jnp.full((K2 - M, N), INF, jnp.float32)], axis=0
        )
        shift = 1
        while shift < N:
            rolled = pltpu.roll(buf, shift, 0)
            buf = jnp.where((lane2 & shift) != 0, rolled, buf)
            shift *= 2
        skb_ref[:, CH * g : CH * (g + 1), :] = buf.reshape(K2, CH, LW)

    lane1 = lax.broadcasted_iota(jnp.int32, (G, N), 1)

    def shift1(p):
        r = pltpu.roll(p, 1, 1)
        return jnp.where(lane1 == 0, INF, r)

    def page(k_dyn):
        return skb_ref[pl.ds(k_dyn, 1), :, :].reshape(G, N)

    cd0 = page(0)
    p1 = jnp.where(lane1 == 0, cd0, INF)
    st_ref[0] = p1
    st_ref[1] = shift1(p1)
    st_ref[2] = jnp.full((G, N), INF, jnp.float32)

    def dp_body(k, carry):
        cd = page(k)
        c = st_ref[0]
        b = st_ref[pl.ds(1 + lax.rem(k + 1, 2), 1)].reshape(G, N)
        a_pg = 1 + lax.rem(k, 2)
        a = st_ref[pl.ds(a_pg, 1)].reshape(G, N)
        mbc = jnp.minimum(b, c)
        v = cd + jnp.minimum(a, mbc)
        diag = a <= mbc
        up = jnp.logical_not(diag) & (b <= c)
        code = jnp.where(diag, 0.0, jnp.where(up, 4.0, 8.0))
        skb_ref[pl.ds(k, 1), :, :] = (cd + code).reshape(1, CH * G, LW)
        st_ref[0] = v
        st_ref[pl.ds(a_pg, 1)] = shift1(v).reshape(1, G, N)
        return carry

    lax.fori_loop(1, K, dp_body, 0, unroll=2)

    laneM = lax.broadcasted_iota(jnp.int32, (1, M), 1)
    skew00 = [skb_ref[0, CH * g, 0] for g in range(G)]

    def _dec(e):
        return (
            e
            - 4.0 * (e >= 3.0).astype(jnp.float32)
            - 4.0 * (e >= 7.0).astype(jnp.float32)
        )

    c_last = [
        _dec(skb_ref[K - 1, CH * g + CH - 1, LW - 1]) for g in range(G)
    ]

    for g in range(G):
        cs_ref[g : g + 1, :] = jnp.where(
            laneM == (M - 1), c_last[g], jnp.float32(0.0)
        )

    def read_enc(g, k, i):
        sub = CH * g + lax.div(i, LW)
        off = lax.rem(i, LW)
        row = skb_ref[pl.ds(k, 1), pl.ds(sub, 1), :]
        amt = lax.rem(jnp.int32(LW) - off, jnp.int32(LW))
        return pltpu.roll(row, amt, 2)[0, 0, 0]

    iN = jnp.int32(N - 1)
    jM = jnp.int32(M - 1)
    enc0 = [read_enc(g, K - 1, iN) for g in range(G)]
    ij0 = iN * jnp.int32(1024) + jM
    init = tuple([ij0] * G + enc0)

    def bt_body(t, carry):
        ijs = list(carry[:G])
        encs = list(carry[G:])
        for g in range(G):
            ij, enc = ijs[g], encs[g]
            i = lax.shift_right_logical(ij, 10)
            j = lax.bitwise_and(ij, jnp.int32(1023))
            pred = (i > 0) & (j > 0)
            ge4 = enc >= 3.0
            ge8 = enc >= 7.0
            go_i = pred & jnp.logical_not(ge8)
            go_j = pred & (jnp.logical_not(ge4) | ge8)
            nij = (
                ij
                - jnp.where(go_i, jnp.int32(1024), jnp.int32(0))
                - jnp.where(go_j, jnp.int32(1), jnp.int32(0))
            )
            ni = lax.shift_right_logical(nij, 10)
            nj = lax.bitwise_and(nij, jnp.int32(1023))
            kn = ni + nj
            enc_n = jnp.where(kn == 0, skew00[g], read_enc(g, kn, ni))
            cost_n = _dec(enc_n)
            cs_ref[g : g + 1, :] = cs_ref[g : g + 1, :] + jnp.where(
                pred & (laneM == nj), cost_n, jnp.float32(0.0)
            )
            ijs[g], encs[g] = nij, enc_n
        return tuple(ijs + encs)

    fin = lax.fori_loop(0, K - 1, bt_body, init, unroll=2)

    rows = []
    for g in range(G):
        both0 = fin[g] == 0
        cs = cs_ref[g : g + 1, :] + jnp.where(
            (laneM == 0) & jnp.logical_not(both0), skew00[g], jnp.float32(0.0)
        )
        mpos = jnp.max(cs)
        pos = mpos + jnp.log(jnp.sum(jnp.exp(cs - mpos)))
        rows.append(jnp.full((1, 128), pos - neg_v[g, 0], jnp.float32))
    out_ref[0] = jnp.concatenate(rows, axis=0)


def _dtw_pallas(x, y, interpret=False):
    B, N, D = x.shape
    M = y.shape[1]
    G = _G
    assert B % G == 0 and N < 1024 and M < 1024
    K2 = ((N + M - 1) + 7) // 8 * 8
    LW = 128 if N % 128 == 0 else N
    CH = N // LW
    xg = x.reshape(B // G, G, N, D)
    yg = y.reshape(B // G, G, M, D)
    out = pl.pallas_call(
        _dtw_body,
        grid=(B // G,),
        in_specs=[
            pl.BlockSpec((1, G, N, D), lambda b: (b, 0, 0, 0)),
            pl.BlockSpec((1, G, M, D), lambda b: (b, 0, 0, 0)),
        ],
        out_specs=pl.BlockSpec((1, G, 128), lambda b: (b, 0, 0)),
        out_shape=jax.ShapeDtypeStruct((B // G, G, 128), jnp.float32),
        scratch_shapes=[
            pltpu.VMEM((3, G, N), jnp.float32),
            pltpu.VMEM((G, M), jnp.float32),
            pltpu.VMEM((K2, CH * G, LW), jnp.float32),
        ],
        compiler_params=pltpu.CompilerParams(
            dimension_semantics=("parallel",),
            vmem_limit_bytes=48 * 1024 * 1024,
        ),
        interpret=interpret,
    )(xg, yg)
    return out[:, :, 0].reshape(B)


def kernel(x, y):
    return _dtw_pallas(x, y)


# speedup vs baseline: 1.1770x; 1.0451x over previous
import jax
import jax.numpy as jnp
from jax import lax
from jax.experimental import pallas as pl
from jax.experimental.pallas import tpu as pltpu

_EPS = 1e-8
_G = 8


def _dtw_body(x_ref, y_ref, out_ref, st_ref, cs_ref, skb_ref):
    G = x_ref.shape[1]
    N = x_ref.shape[2]
    M = y_ref.shape[2]
    K = N + M - 1
    K2 = skb_ref.shape[0]
    LW = skb_ref.shape[2]
    CH = N // LW
    INF = jnp.float32(jnp.inf)

    x = x_ref[0]
    y = y_ref[0]
    xn = x / jnp.maximum(jnp.sqrt(jnp.sum(x * x, axis=2, keepdims=True)), _EPS)
    yn = y / jnp.maximum(jnp.sqrt(jnp.sum(y * y, axis=2, keepdims=True)), _EPS)
    costT = 1.0 - lax.dot_general(
        yn, xn, (((2,), (2,)), ((0,), (0,))),
        preferred_element_type=jnp.float32,
    )

    s_all = jnp.sum(costT, axis=2)
    mneg = jnp.max(s_all, axis=1, keepdims=True)
    neg_v = mneg + jnp.log(
        jnp.sum(jnp.exp(s_all - mneg), axis=1, keepdims=True)
    )

    lane2 = lax.broadcasted_iota(jnp.int32, (K2, N), 1)
    for g in range(G):
        buf = jnp.concatenate(
            [costT[g], jnp.full((K2 - M, N), INF, jnp.float32)], axis=0
        )
        shift = 1
        while shift < N:
            rolled = pltpu.roll(buf, shift, 0)
            buf = jnp.where((lane2 & shift) != 0, rolled, buf)
            shift *= 2
        skb_ref[:, CH * g : CH * (g + 1), :] = buf.reshape(K2, CH, LW)

    lane1 = lax.broadcasted_iota(jnp.int32, (G, N), 1)

    def shift1(p):
        r = pltpu.roll(p, 1, 1)
        return jnp.where(lane1 == 0, INF, r)

    def page(k_dyn):
        return skb_ref[pl.ds(k_dyn, 1), :, :].reshape(G, N)

    cd0 = page(0)
    p1 = jnp.where(lane1 == 0, cd0, INF)
    st_ref[0] = p1
    st_ref[1] = shift1(p1)
    st_ref[2] = jnp.full((G, N), INF, jnp.float32)

    def dp_body(k, carry):
        cd = page(k)
        c = st_ref[0]
        b = st_ref[pl.ds(1 + lax.rem(k + 1, 2), 1)].reshape(G, N)
        a_pg = 1 + lax.rem(k, 2)
        a = st_ref[pl.ds(a_pg, 1)].reshape(G, N)
        mbc = jnp.minimum(b, c)
        v = cd + jnp.minimum(a, mbc)
        diag = a <= mbc
        up = jnp.logical_not(diag) & (b <= c)
        code = jnp.where(diag, 0.0, jnp.where(up, 4.0, 8.0))
        skb_ref[pl.ds(k, 1), :, :] = (cd + code).reshape(1, CH * G, LW)
        st_ref[0] = v
        st_ref[pl.ds(a_pg, 1)] = shift1(v).reshape(1, G, N)
        return carry

    lax.fori_loop(1, K, dp_body, 0, unroll=4)

    laneM = lax.broadcasted_iota(jnp.int32, (1, M), 1)
    skew00 = [skb_ref[0, CH * g, 0] for g in range(G)]

    def _dec(e):
        return (
            e
            - 4.0 * (e >= 3.0).astype(jnp.float32)
            - 4.0 * (e >= 7.0).astype(jnp.float32)
        )

    c_last = [
        _dec(skb_ref[K - 1, CH * g + CH - 1, LW - 1]) for g in range(G)
    ]

    for g in range(G):
        cs_ref[g : g + 1, :] = jnp.where(
            laneM == (M - 1), c_last[g], jnp.float32(0.0)
        )

    def read_enc(g, k, i):
        sub = CH * g + lax.div(i, LW)
        off = lax.rem(i, LW)
        row = skb_ref[pl.ds(k, 1), pl.ds(sub, 1), :]
        amt = lax.rem(jnp.int32(LW) - off, jnp.int32(LW))
        return pltpu.roll(row, amt, 2)[0, 0, 0]

    iN = jnp.int32(N - 1)
    jM = jnp.int32(M - 1)
    enc0 = [read_enc(g, K - 1, iN) for g in range(G)]
    ij0 = iN * jnp.int32(1024) + jM
    init = tuple([ij0] * G + enc0)

    def bt_body(t, carry):
        ijs = list(carry[:G])
        encs = list(carry[G:])
        for g in range(G):
            ij, enc = ijs[g], encs[g]
            i = lax.shift_right_logical(ij, 10)
            j = lax.bitwise_and(ij, jnp.int32(1023))
            pred = (i > 0) & (j > 0)
            ge4 = enc >= 3.0
            ge8 = enc >= 7.0
            go_i = pred & jnp.logical_not(ge8)
            go_j = pred & (jnp.logical_not(ge4) | ge8)
            nij = (
                ij
                - jnp.where(go_i, jnp.int32(1024), jnp.int32(0))
                - jnp.where(go_j, jnp.int32(1), jnp.int32(0))
            )
            ni = lax.shift_right_logical(nij, 10)
            nj = lax.bitwise_and(nij, jnp.int32(1023))
            kn = ni + nj
            enc_n = jnp.where(kn == 0, skew00[g], read_enc(g, kn, ni))
            cost_n = _dec(enc_n)
            cs_ref[g : g + 1, :] = cs_ref[g : g + 1, :] + jnp.where(
                pred & (laneM == nj), cost_n, jnp.float32(0.0)
            )
            ijs[g], encs[g] = nij, enc_n
        return tuple(ijs + encs)

    fin = lax.fori_loop(0, K - 1, bt_body, init, unroll=4)

    rows = []
    for g in range(G):
        both0 = fin[g] == 0
        cs = cs_ref[g : g + 1, :] + jnp.where(
            (laneM == 0) & jnp.logical_not(both0), skew00[g], jnp.float32(0.0)
        )
        mpos = jnp.max(cs)
        pos = mpos + jnp.log(jnp.sum(jnp.exp(cs - mpos)))
        rows.append(jnp.full((1, 128), pos - neg_v[g, 0], jnp.float32))
    out_ref[0] = jnp.concatenate(rows, axis=0)


def _dtw_pallas(x, y, interpret=False):
    B, N, D = x.shape
    M = y.shape[1]
    G = _G
    assert B % G == 0 and N < 1024 and M < 1024
    K2 = ((N + M - 1) + 7) // 8 * 8
    LW = 128 if N % 128 == 0 else N
    CH = N // LW
    xg = x.reshape(B // G, G, N, D)
    yg = y.reshape(B // G, G, M, D)
    out = pl.pallas_call(
        _dtw_body,
        grid=(B // G,),
        in_specs=[
            pl.BlockSpec((1, G, N, D), lambda b: (b, 0, 0, 0)),
            pl.BlockSpec((1, G, M, D), lambda b: (b, 0, 0, 0)),
        ],
        out_specs=pl.BlockSpec((1, G, 128), lambda b: (b, 0, 0)),
        out_shape=jax.ShapeDtypeStruct((B // G, G, 128), jnp.float32),
        scratch_shapes=[
            pltpu.VMEM((3, G, N), jnp.float32),
            pltpu.VMEM((G, M), jnp.float32),
            pltpu.VMEM((K2, CH * G, LW), jnp.float32),
        ],
        compiler_params=pltpu.CompilerParams(
            dimension_semantics=("parallel",),
            vmem_limit_bytes=48 * 1024 * 1024,
        ),
        interpret=interpret,
    )(xg, yg)
    return out[:, :, 0].reshape(B)


def kernel(x, y):
    return _dtw_pallas(x, y)
